# baseline (speedup 1.0000x reference)
import jax
import jax.numpy as jnp
from jax import lax
from jax.experimental import pallas as pl
from jax.experimental.pallas import tpu as pltpu, tpu_sc as plsc

N = 10000
E = 320000
D_IN = 128
D_HID = 41
D_OUT = 32
DP = 48

NC, NS = 2, 16
NW = NC * NS
EPW = E // NW
CHUNK = 80
NCHUNK = EPW // CHUNK
RPT = N // NS
ZROWS = 125

_f32 = jnp.float32


NBLK = 10
BLK = N // NBLK


def _make_head_body(one_col):
    def body(x_ref, w_ref, al_ref, ar_ref, htab_ref, el_ref, er_ref):
        h = jnp.dot(x_ref[...], w_ref[...], preferred_element_type=_f32)
        el_ref[...] = jnp.sum(h * al_ref[...][None, :], axis=1, keepdims=True)
        er_ref[...] = jnp.sum(h * ar_ref[...][None, :], axis=1, keepdims=True)
        col = lax.broadcasted_iota(jnp.int32, h.shape, 1)
        htab_ref[...] = jnp.where(col == one_col, 1.0, h)
    return body


def _tc_head1(x, w1p, al1p, ar1p):
    return pl.pallas_call(
        _make_head_body(D_HID),
        grid=(NBLK,),
        in_specs=[
            pl.BlockSpec((BLK, D_IN), lambda i: (i, 0)),
            pl.BlockSpec((D_IN, DP), lambda i: (0, 0)),
            pl.BlockSpec((DP,), lambda i: (0,)),
            pl.BlockSpec((DP,), lambda i: (0,)),
        ],
        out_specs=[
            pl.BlockSpec((BLK, DP), lambda i: (i, 0)),
            pl.BlockSpec((BLK, 1), lambda i: (i, 0)),
            pl.BlockSpec((BLK, 1), lambda i: (i, 0)),
        ],
        out_shape=[
            jax.ShapeDtypeStruct((N, DP), _f32),
            jax.ShapeDtypeStruct((N, 1), _f32),
            jax.ShapeDtypeStruct((N, 1), _f32),
        ],
    )(x, w1p, al1p, ar1p)


def _tc_mid_body(num_ref, b_ref, w_ref, al_ref, ar_ref,
                 htab_ref, el_ref, er_ref):
    ns = num_ref[0] + num_ref[1]
    den = ns[:, D_HID:D_HID + 1]
    den = jnp.where(den == 0.0, 1.0, den)
    x1 = ns / den + b_ref[...][None, :]
    a = jnp.where(x1 > 0.0, x1, jnp.exp(x1) - 1.0)
    h = jnp.dot(a, w_ref[...], preferred_element_type=_f32)
    el_ref[...] = jnp.sum(h * al_ref[...][None, :], axis=1, keepdims=True)
    er_ref[...] = jnp.sum(h * ar_ref[...][None, :], axis=1, keepdims=True)
    col = lax.broadcasted_iota(jnp.int32, h.shape, 1)
    htab_ref[...] = jnp.where(col == D_OUT, 1.0, h)


def _tc_mid(num1, b1p, w2p, al2p, ar2p):
    return pl.pallas_call(
        _tc_mid_body,
        grid=(NBLK,),
        in_specs=[
            pl.BlockSpec((NC, BLK, DP), lambda i: (0, i, 0)),
            pl.BlockSpec((DP,), lambda i: (0,)),
            pl.BlockSpec((DP, DP), lambda i: (0, 0)),
            pl.BlockSpec((DP,), lambda i: (0,)),
            pl.BlockSpec((DP,), lambda i: (0,)),
        ],
        out_specs=[
            pl.BlockSpec((BLK, DP), lambda i: (i, 0)),
            pl.BlockSpec((BLK, 1), lambda i: (i, 0)),
            pl.BlockSpec((BLK, 1), lambda i: (i, 0)),
        ],
        out_shape=[
            jax.ShapeDtypeStruct((N, DP), _f32),
            jax.ShapeDtypeStruct((N, 1), _f32),
            jax.ShapeDtypeStruct((N, 1), _f32),
        ],
    )(num1, b1p, w2p, al2p, ar2p)


def _tc_final_body(num_ref, b_ref, out_ref):
    ns = num_ref[0] + num_ref[1]
    den = ns[:, D_OUT:D_OUT + 1]
    den = jnp.where(den == 0.0, 1.0, den)
    out_ref[...] = ns[:, :D_OUT] / den + b_ref[...][None, :]


def _tc_final(num2, b2):
    return pl.pallas_call(
        _tc_final_body,
        grid=(NBLK,),
        in_specs=[
            pl.BlockSpec((NC, BLK, DP), lambda i: (0, i, 0)),
            pl.BlockSpec((D_OUT,), lambda i: (0,)),
        ],
        out_specs=pl.BlockSpec((BLK, D_OUT), lambda i: (i, 0)),
        out_shape=jax.ShapeDtypeStruct((N, D_OUT), _f32),
    )(num2, b2)


def _sc_edge_body(ei_hbm, htab_hbm, el_hbm, er_hbm, num_hbm,
                  src_v, dst_v, el_v, er_v, rows0_v, rows1_v, rows2_v,
                  zbuf_v, acc_sh, htab_sh,
                  gsem0, gsem1, gsem2, ssem0, ssem1, ssem2):
    cid = lax.axis_index("c")
    tid = lax.axis_index("s")
    wid = tid * NC + cid

    z16 = jnp.zeros((16,), _f32)

    def zloop(i, c):
        zbuf_v[i, pl.ds(0, 16)] = z16
        zbuf_v[i, pl.ds(16, 16)] = z16
        zbuf_v[i, pl.ds(32, 16)] = z16
        return c

    lax.fori_loop(0, ZROWS, zloop, 0)
    for k in range(RPT // ZROWS):
        pltpu.sync_copy(zbuf_v,
                        acc_sh.at[pl.ds(tid * RPT + k * ZROWS, ZROWS)])
    pltpu.sync_copy(htab_hbm.at[pl.ds(tid * RPT, RPT)],
                    htab_sh.at[pl.ds(tid * RPT, RPT)])
    plsc.subcore_barrier()

    pltpu.sync_copy(ei_hbm.at[0, wid], src_v)
    pltpu.sync_copy(ei_hbm.at[1, wid], dst_v)
    pltpu.sync_copy(el_hbm, el_v)
    pltpu.sync_copy(er_hbm, er_v)

    bufs = (rows0_v, rows1_v, rows2_v)
    gsems = (gsem0, gsem1, gsem2)
    ssems = (ssem0, ssem1, ssem2)

    def start_gather(j, b):
        pltpu.async_copy(htab_sh.at[src_v.at[j]], bufs[b], gsems[b])

    def wait_gather(j, b):
        pltpu.make_async_copy(htab_sh.at[src_v.at[j]], bufs[b],
                              gsems[b]).wait()

    def start_scatter(j, b):
        pltpu.async_copy(bufs[b], acc_sh.at[dst_v.at[j]], ssems[b], add=True)

    def wait_scatter(j, b):
        pltpu.make_async_copy(bufs[b], acc_sh.at[dst_v.at[j]],
                              ssems[b]).wait()

    def scale(j, b):
        buf = bufs[b]
        for g in range(CHUNK // 16):
            sl = pl.ds(g * 16, 16)
            isv = src_v[j, sl]
            idv = dst_v[j, sl]
            xe = plsc.load_gather(el_v, [isv]) + plsc.load_gather(er_v, [idv])
            e = jnp.maximum(xe, 0.2 * xe)
            s = jnp.exp(e)
            for l in range(16):
                i = g * 16 + l
                sc = s[l]
                buf[i, pl.ds(0, 16)] = buf[i, pl.ds(0, 16)] * sc
                buf[i, pl.ds(16, 16)] = buf[i, pl.ds(16, 16)] * sc
                buf[i, pl.ds(32, 16)] = buf[i, pl.ds(32, 16)] * sc

    start_gather(0, 0)
    start_gather(1, 1)
    wait_gather(0, 0)
    scale(0, 0)
    start_gather(2, 2)
    start_scatter(0, 0)

    def triple(i, c):
        base = 3 * i
        for off in range(1, 4):
            ch = base + off
            b = off % 3
            bp = (off - 1) % 3
            wait_gather(ch, b)
            scale(ch, b)
            wait_scatter(ch - 1, bp)

            @pl.when(ch + 2 < NCHUNK)
            def _():
                start_gather(ch + 2, bp)

            start_scatter(ch, b)
        return c

    lax.fori_loop(0, (NCHUNK - 2) // 3, triple, 0)
    last = NCHUNK - 1
    wait_gather(last, 1)
    scale(last, 1)
    wait_scatter(last - 1, 0)
    start_scatter(last, 1)
    wait_scatter(last, 1)

    plsc.subcore_barrier()
    pltpu.sync_copy(acc_sh.at[pl.ds(tid * RPT, RPT)], num_hbm.at[cid, tid])


_sc_mesh = plsc.VectorSubcoreMesh(
    core_axis_name="c", subcore_axis_name="s",
    num_cores=NC, num_subcores=NS)

_sc_edge = pl.kernel(
    _sc_edge_body,
    out_type=jax.ShapeDtypeStruct((NC, NS, RPT, DP), _f32),
    mesh=_sc_mesh,
    compiler_params=pltpu.CompilerParams(needs_layout_passes=False,
                                         use_tc_tiling_on_sc=False),
    scratch_types=[
        pltpu.VMEM((NCHUNK, CHUNK), jnp.int32),
        pltpu.VMEM((NCHUNK, CHUNK), jnp.int32),
        pltpu.VMEM((N,), _f32),
        pltpu.VMEM((N,), _f32),
        pltpu.VMEM((CHUNK, DP), _f32),
        pltpu.VMEM((CHUNK, DP), _f32),
        pltpu.VMEM((CHUNK, DP), _f32),
        pltpu.VMEM((ZROWS, DP), _f32),
        pltpu.VMEM_SHARED((N, DP), _f32),
        pltpu.VMEM_SHARED((N, DP), _f32),
        pltpu.SemaphoreType.DMA,
        pltpu.SemaphoreType.DMA,
        pltpu.SemaphoreType.DMA,
        pltpu.SemaphoreType.DMA,
        pltpu.SemaphoreType.DMA,
        pltpu.SemaphoreType.DMA,
    ],
)


def kernel(features, edge_index, W1, al1, ar1, b1, W2, al2, ar2, b2):
    ei4 = edge_index.reshape(2, NW, NCHUNK, CHUNK)
    w1p = jnp.zeros((D_IN, DP), _f32).at[:, :D_HID].set(W1)
    al1p = jnp.zeros((DP,), _f32).at[:D_HID].set(al1)
    ar1p = jnp.zeros((DP,), _f32).at[:D_HID].set(ar1)
    b1p = jnp.zeros((DP,), _f32).at[:D_HID].set(b1)
    w2p = jnp.zeros((DP, DP), _f32).at[:D_HID, :D_OUT].set(W2)
    al2p = jnp.zeros((DP,), _f32).at[:D_OUT].set(al2)
    ar2p = jnp.zeros((DP,), _f32).at[:D_OUT].set(ar2)

    htab1, el1, er1 = _tc_head1(features, w1p, al1p, ar1p)
    num1 = _sc_edge(ei4, htab1, el1.reshape(N),
                    er1.reshape(N)).reshape(NC, N, DP)
    htab2, el2, er2 = _tc_mid(num1, b1p, w2p, al2p, ar2p)
    num2 = _sc_edge(ei4, htab2, el2.reshape(N),
                    er2.reshape(N)).reshape(NC, N, DP)
    return _tc_final(num2, b2)


# speedup vs baseline: 1.0881x; 1.0881x over previous
import jax
import jax.numpy as jnp
from jax import lax
from jax.experimental import pallas as pl
from jax.experimental.pallas import tpu as pltpu, tpu_sc as plsc

N = 10000
E = 320000
D_IN = 128
D_HID = 41
D_OUT = 32
DP = 48

NC, NS = 2, 16
NW = NC * NS
EPW = E // NW
CHUNK = 80
NCHUNK = EPW // CHUNK
RPT = N // NS
ZROWS = 125

_f32 = jnp.float32


NBLK = 10
BLK = N // NBLK


def _make_head_body(one_col):
    def body(x_ref, w_ref, al_ref, ar_ref, htab_ref, el_ref, er_ref):
        h = jnp.dot(x_ref[...], w_ref[...], preferred_element_type=_f32)
        el_ref[...] = jnp.sum(h * al_ref[...][None, :], axis=1)
        er_ref[...] = jnp.sum(h * ar_ref[...][None, :], axis=1)
        col = lax.broadcasted_iota(jnp.int32, h.shape, 1)
        htab_ref[...] = jnp.where(col == one_col, 1.0, h)
    return body


def _tc_head1(x, w1p, al1p, ar1p):
    return pl.pallas_call(
        _make_head_body(D_HID),
        out_shape=[
            jax.ShapeDtypeStruct((N, DP), _f32),
            jax.ShapeDtypeStruct((N,), _f32),
            jax.ShapeDtypeStruct((N,), _f32),
        ],
    )(x, w1p, al1p, ar1p)


def _tc_mid_body(num_ref, b_ref, w_ref, al_ref, ar_ref,
                 htab_ref, el_ref, er_ref):
    ns = num_ref[0] + num_ref[1]
    den = ns[:, D_HID:D_HID + 1]
    den = jnp.where(den == 0.0, 1.0, den)
    x1 = ns / den + b_ref[...][None, :]
    a = jnp.where(x1 > 0.0, x1, jnp.exp(x1) - 1.0)
    h = jnp.dot(a, w_ref[...], preferred_element_type=_f32)
    el_ref[...] = jnp.sum(h * al_ref[...][None, :], axis=1)
    er_ref[...] = jnp.sum(h * ar_ref[...][None, :], axis=1)
    col = lax.broadcasted_iota(jnp.int32, h.shape, 1)
    htab_ref[...] = jnp.where(col == D_OUT, 1.0, h)


def _tc_mid(num1, b1p, w2p, al2p, ar2p):
    return pl.pallas_call(
        _tc_mid_body,
        out_shape=[
            jax.ShapeDtypeStruct((N, DP), _f32),
            jax.ShapeDtypeStruct((N,), _f32),
            jax.ShapeDtypeStruct((N,), _f32),
        ],
    )(num1, b1p, w2p, al2p, ar2p)


def _tc_final_body(num_ref, b_ref, out_ref):
    ns = num_ref[0] + num_ref[1]
    den = ns[:, D_OUT:D_OUT + 1]
    den = jnp.where(den == 0.0, 1.0, den)
    out_ref[...] = ns[:, :D_OUT] / den + b_ref[...][None, :]


def _tc_final(num2, b2):
    return pl.pallas_call(
        _tc_final_body,
        out_shape=jax.ShapeDtypeStruct((N, D_OUT), _f32),
    )(num2, b2)


def _sc_edge_body(ei_hbm, htab_hbm, el_hbm, er_hbm, num_hbm,
                  src_v, dst_v, el_v, er_v, rows0_v, rows1_v, rows2_v,
                  zbuf_v, acc_sh, htab_sh,
                  gsem0, gsem1, gsem2, ssem0, ssem1, ssem2):
    cid = lax.axis_index("c")
    tid = lax.axis_index("s")
    wid = tid * NC + cid

    z16 = jnp.zeros((16,), _f32)

    def zloop(i, c):
        zbuf_v[i, pl.ds(0, 16)] = z16
        zbuf_v[i, pl.ds(16, 16)] = z16
        zbuf_v[i, pl.ds(32, 16)] = z16
        return c

    lax.fori_loop(0, ZROWS, zloop, 0)
    for k in range(RPT // ZROWS):
        pltpu.sync_copy(zbuf_v,
                        acc_sh.at[pl.ds(tid * RPT + k * ZROWS, ZROWS)])
    pltpu.sync_copy(htab_hbm.at[pl.ds(tid * RPT, RPT)],
                    htab_sh.at[pl.ds(tid * RPT, RPT)])
    plsc.subcore_barrier()

    pltpu.sync_copy(ei_hbm.at[0, wid], src_v)
    pltpu.sync_copy(ei_hbm.at[1, wid], dst_v)
    pltpu.sync_copy(el_hbm, el_v)
    pltpu.sync_copy(er_hbm, er_v)

    bufs = (rows0_v, rows1_v, rows2_v)
    gsems = (gsem0, gsem1, gsem2)
    ssems = (ssem0, ssem1, ssem2)

    def start_gather(j, b):
        pltpu.async_copy(htab_sh.at[src_v.at[j]], bufs[b], gsems[b])

    def wait_gather(j, b):
        pltpu.make_async_copy(htab_sh.at[src_v.at[j]], bufs[b],
                              gsems[b]).wait()

    def start_scatter(j, b):
        pltpu.async_copy(bufs[b], acc_sh.at[dst_v.at[j]], ssems[b], add=True)

    def wait_scatter(j, b):
        pltpu.make_async_copy(bufs[b], acc_sh.at[dst_v.at[j]],
                              ssems[b]).wait()

    def scale(j, b):
        buf = bufs[b]
        for g in range(CHUNK // 16):
            sl = pl.ds(g * 16, 16)
            isv = src_v[j, sl]
            idv = dst_v[j, sl]
            xe = plsc.load_gather(el_v, [isv]) + plsc.load_gather(er_v, [idv])
            e = jnp.maximum(xe, 0.2 * xe)
            s = jnp.exp(e)
            for l in range(16):
                i = g * 16 + l
                sc = s[l]
                buf[i, pl.ds(0, 16)] = buf[i, pl.ds(0, 16)] * sc
                buf[i, pl.ds(16, 16)] = buf[i, pl.ds(16, 16)] * sc
                buf[i, pl.ds(32, 16)] = buf[i, pl.ds(32, 16)] * sc

    start_gather(0, 0)
    start_gather(1, 1)
    wait_gather(0, 0)
    scale(0, 0)
    start_gather(2, 2)
    start_scatter(0, 0)

    def triple(i, c):
        base = 3 * i
        for off in range(1, 4):
            ch = base + off
            b = off % 3
            bp = (off - 1) % 3
            wait_gather(ch, b)
            scale(ch, b)
            wait_scatter(ch - 1, bp)

            @pl.when(ch + 2 < NCHUNK)
            def _():
                start_gather(ch + 2, bp)

            start_scatter(ch, b)
        return c

    lax.fori_loop(0, (NCHUNK - 2) // 3, triple, 0)
    last = NCHUNK - 1
    wait_gather(last, 1)
    scale(last, 1)
    wait_scatter(last - 1, 0)
    start_scatter(last, 1)
    wait_scatter(last, 1)

    plsc.subcore_barrier()
    pltpu.sync_copy(acc_sh.at[pl.ds(tid * RPT, RPT)], num_hbm.at[cid, tid])


_sc_mesh = plsc.VectorSubcoreMesh(
    core_axis_name="c", subcore_axis_name="s",
    num_cores=NC, num_subcores=NS)

_sc_edge = pl.kernel(
    _sc_edge_body,
    out_type=jax.ShapeDtypeStruct((NC, NS, RPT, DP), _f32),
    mesh=_sc_mesh,
    compiler_params=pltpu.CompilerParams(needs_layout_passes=False,
                                         use_tc_tiling_on_sc=False),
    scratch_types=[
        pltpu.VMEM((NCHUNK, CHUNK), jnp.int32),
        pltpu.VMEM((NCHUNK, CHUNK), jnp.int32),
        pltpu.VMEM((N,), _f32),
        pltpu.VMEM((N,), _f32),
        pltpu.VMEM((CHUNK, DP), _f32),
        pltpu.VMEM((CHUNK, DP), _f32),
        pltpu.VMEM((CHUNK, DP), _f32),
        pltpu.VMEM((ZROWS, DP), _f32),
        pltpu.VMEM_SHARED((N, DP), _f32),
        pltpu.VMEM_SHARED((N, DP), _f32),
        pltpu.SemaphoreType.DMA,
        pltpu.SemaphoreType.DMA,
        pltpu.SemaphoreType.DMA,
        pltpu.SemaphoreType.DMA,
        pltpu.SemaphoreType.DMA,
        pltpu.SemaphoreType.DMA,
    ],
)


def kernel(features, edge_index, W1, al1, ar1, b1, W2, al2, ar2, b2):
    ei4 = edge_index.reshape(2, NW, NCHUNK, CHUNK)
    w1p = jnp.zeros((D_IN, DP), _f32).at[:, :D_HID].set(W1)
    al1p = jnp.zeros((DP,), _f32).at[:D_HID].set(al1)
    ar1p = jnp.zeros((DP,), _f32).at[:D_HID].set(ar1)
    b1p = jnp.zeros((DP,), _f32).at[:D_HID].set(b1)
    w2p = jnp.zeros((DP, DP), _f32).at[:D_HID, :D_OUT].set(W2)
    al2p = jnp.zeros((DP,), _f32).at[:D_OUT].set(al2)
    ar2p = jnp.zeros((DP,), _f32).at[:D_OUT].set(ar2)

    htab1, el1, er1 = _tc_head1(features, w1p, al1p, ar1p)
    num1 = _sc_edge(ei4, htab1, el1, er1).reshape(NC, N, DP)
    htab2, el2, er2 = _tc_mid(num1, b1p, w2p, al2p, ar2p)
    num2 = _sc_edge(ei4, htab2, el2, er2).reshape(NC, N, DP)
    return _tc_final(num2, b2)


# speedup vs baseline: 1.1282x; 1.0368x over previous
import jax
import jax.numpy as jnp
from jax import lax
from jax.experimental import pallas as pl
from jax.experimental.pallas import tpu as pltpu, tpu_sc as plsc

N = 10000
E = 320000
D_IN = 128
D_HID = 41
D_OUT = 32
DP = 48

NC, NS = 2, 16
NW = NC * NS
EPW = E // NW
CHUNK = 80
NCHUNK = EPW // CHUNK
RPT = N // NS
ZROWS = 125

_f32 = jnp.float32


def _make_head_body(one_col):
    def body(x_ref, w_ref, al_ref, ar_ref, htab_ref, el_ref, er_ref):
        h = jnp.dot(x_ref[...], w_ref[...], preferred_element_type=_f32)
        el_ref[...] = jnp.sum(h * al_ref[...][None, :], axis=1)
        er_ref[...] = jnp.sum(h * ar_ref[...][None, :], axis=1)
        col = lax.broadcasted_iota(jnp.int32, h.shape, 1)
        htab_ref[...] = jnp.where(col == one_col, 1.0, h)
    return body


def _tc_head1(x, w1p, al1p, ar1p):
    return pl.pallas_call(
        _make_head_body(D_HID),
        out_shape=[
            jax.ShapeDtypeStruct((N, DP), _f32),
            jax.ShapeDtypeStruct((N,), _f32),
            jax.ShapeDtypeStruct((N,), _f32),
        ],
    )(x, w1p, al1p, ar1p)


def _tc_mid_body(num_ref, b_ref, w_ref, al_ref, ar_ref,
                 htab_ref, el_ref, er_ref):
    ns = num_ref[0] + num_ref[1]
    den = ns[:, D_HID:D_HID + 1]
    den = jnp.where(den == 0.0, 1.0, den)
    x1 = ns / den + b_ref[...][None, :]
    a = jnp.where(x1 > 0.0, x1, jnp.exp(x1) - 1.0)
    h = jnp.dot(a, w_ref[...], preferred_element_type=_f32)
    el_ref[...] = jnp.sum(h * al_ref[...][None, :], axis=1)
    er_ref[...] = jnp.sum(h * ar_ref[...][None, :], axis=1)
    col = lax.broadcasted_iota(jnp.int32, h.shape, 1)
    htab_ref[...] = jnp.where(col == D_OUT, 1.0, h)


def _tc_mid(num1, b1p, w2p, al2p, ar2p):
    return pl.pallas_call(
        _tc_mid_body,
        out_shape=[
            jax.ShapeDtypeStruct((N, DP), _f32),
            jax.ShapeDtypeStruct((N,), _f32),
            jax.ShapeDtypeStruct((N,), _f32),
        ],
    )(num1, b1p, w2p, al2p, ar2p)


def _tc_final_body(num_ref, b_ref, out_ref):
    ns = num_ref[0] + num_ref[1]
    den = ns[:, D_OUT:D_OUT + 1]
    den = jnp.where(den == 0.0, 1.0, den)
    out_ref[...] = ns[:, :D_OUT] / den + b_ref[...][None, :]


def _tc_final(num2, b2):
    return pl.pallas_call(
        _tc_final_body,
        out_shape=jax.ShapeDtypeStruct((N, D_OUT), _f32),
    )(num2, b2)


def _sc_edge_body(ei_hbm, htab_hbm, el_hbm, er_hbm, num_hbm,
                  src_v, dst_v, el_v, er_v, rows0_v, rows1_v, rows2_v,
                  zbuf_v, acc_sh, htab_sh,
                  gsem0, gsem1, gsem2, ssem0, ssem1, ssem2):
    cid = lax.axis_index("c")
    tid = lax.axis_index("s")
    wid = tid * NC + cid

    pltpu.async_copy(ei_hbm.at[0, wid], src_v, gsem0)
    pltpu.async_copy(ei_hbm.at[1, wid], dst_v, gsem1)
    pltpu.async_copy(el_hbm, el_v, gsem2)
    pltpu.async_copy(er_hbm, er_v, ssem0)
    pltpu.async_copy(htab_hbm.at[pl.ds(tid * RPT, RPT)],
                     htab_sh.at[pl.ds(tid * RPT, RPT)], ssem1)

    z16 = jnp.zeros((16,), _f32)

    def zloop(i, c):
        zbuf_v[i, pl.ds(0, 16)] = z16
        zbuf_v[i, pl.ds(16, 16)] = z16
        zbuf_v[i, pl.ds(32, 16)] = z16
        return c

    lax.fori_loop(0, ZROWS, zloop, 0)
    for k in range(RPT // ZROWS):
        pltpu.sync_copy(zbuf_v,
                        acc_sh.at[pl.ds(tid * RPT + k * ZROWS, ZROWS)])

    pltpu.make_async_copy(ei_hbm.at[0, wid], src_v, gsem0).wait()
    pltpu.make_async_copy(ei_hbm.at[1, wid], dst_v, gsem1).wait()
    pltpu.make_async_copy(el_hbm, el_v, gsem2).wait()
    pltpu.make_async_copy(er_hbm, er_v, ssem0).wait()
    pltpu.make_async_copy(htab_hbm.at[pl.ds(tid * RPT, RPT)],
                          htab_sh.at[pl.ds(tid * RPT, RPT)], ssem1).wait()
    plsc.subcore_barrier()

    bufs = (rows0_v, rows1_v, rows2_v)
    gsems = (gsem0, gsem1, gsem2)
    ssems = (ssem0, ssem1, ssem2)

    def start_gather(j, b):
        pltpu.async_copy(htab_sh.at[src_v.at[j]], bufs[b], gsems[b])

    def wait_gather(j, b):
        pltpu.make_async_copy(htab_sh.at[src_v.at[j]], bufs[b],
                              gsems[b]).wait()

    def start_scatter(j, b):
        pltpu.async_copy(bufs[b], acc_sh.at[dst_v.at[j]], ssems[b], add=True)

    def wait_scatter(j, b):
        pltpu.make_async_copy(bufs[b], acc_sh.at[dst_v.at[j]],
                              ssems[b]).wait()

    def scale(j, b):
        buf = bufs[b]
        for g in range(CHUNK // 16):
            sl = pl.ds(g * 16, 16)
            isv = src_v[j, sl]
            idv = dst_v[j, sl]
            xe = plsc.load_gather(el_v, [isv]) + plsc.load_gather(er_v, [idv])
            e = jnp.maximum(xe, 0.2 * xe)
            s = jnp.exp(e)
            for l in range(16):
                i = g * 16 + l
                sc = s[l]
                buf[i, pl.ds(0, 16)] = buf[i, pl.ds(0, 16)] * sc
                buf[i, pl.ds(16, 16)] = buf[i, pl.ds(16, 16)] * sc
                buf[i, pl.ds(32, 16)] = buf[i, pl.ds(32, 16)] * sc

    start_gather(0, 0)
    start_gather(1, 1)
    wait_gather(0, 0)
    scale(0, 0)
    start_gather(2, 2)
    start_scatter(0, 0)

    def triple(i, c):
        base = 3 * i
        for off in range(1, 4):
            ch = base + off
            b = off % 3
            bp = (off - 1) % 3
            wait_gather(ch, b)
            scale(ch, b)
            wait_scatter(ch - 1, bp)

            @pl.when(ch + 2 < NCHUNK)
            def _():
                start_gather(ch + 2, bp)

            start_scatter(ch, b)
        return c

    lax.fori_loop(0, (NCHUNK - 2) // 3, triple, 0)
    last = NCHUNK - 1
    wait_gather(last, 1)
    scale(last, 1)
    wait_scatter(last - 1, 0)
    start_scatter(last, 1)
    wait_scatter(last, 1)

    plsc.subcore_barrier()
    pltpu.sync_copy(acc_sh.at[pl.ds(tid * RPT, RPT)], num_hbm.at[cid, tid])


_sc_mesh = plsc.VectorSubcoreMesh(
    core_axis_name="c", subcore_axis_name="s",
    num_cores=NC, num_subcores=NS)

_sc_edge = pl.kernel(
    _sc_edge_body,
    out_type=jax.ShapeDtypeStruct((NC, NS, RPT, DP), _f32),
    mesh=_sc_mesh,
    compiler_params=pltpu.CompilerParams(needs_layout_passes=False,
                                         use_tc_tiling_on_sc=False),
    scratch_types=[
        pltpu.VMEM((NCHUNK, CHUNK), jnp.int32),
        pltpu.VMEM((NCHUNK, CHUNK), jnp.int32),
        pltpu.VMEM((N,), _f32),
        pltpu.VMEM((N,), _f32),
        pltpu.VMEM((CHUNK, DP), _f32),
        pltpu.VMEM((CHUNK, DP), _f32),
        pltpu.VMEM((CHUNK, DP), _f32),
        pltpu.VMEM((ZROWS, DP), _f32),
        pltpu.VMEM_SHARED((N, DP), _f32),
        pltpu.VMEM_SHARED((N, DP), _f32),
        pltpu.SemaphoreType.DMA,
        pltpu.SemaphoreType.DMA,
        pltpu.SemaphoreType.DMA,
        pltpu.SemaphoreType.DMA,
        pltpu.SemaphoreType.DMA,
        pltpu.SemaphoreType.DMA,
    ],
)


def kernel(features, edge_index, W1, al1, ar1, b1, W2, al2, ar2, b2):
    ei4 = edge_index.reshape(2, NW, NCHUNK, CHUNK)
    w1p = jnp.zeros((D_IN, DP), _f32).at[:, :D_HID].set(W1)
    al1p = jnp.zeros((DP,), _f32).at[:D_HID].set(al1)
    ar1p = jnp.zeros((DP,), _f32).at[:D_HID].set(ar1)
    b1p = jnp.zeros((DP,), _f32).at[:D_HID].set(b1)
    w2p = jnp.zeros((DP, DP), _f32).at[:D_HID, :D_OUT].set(W2)
    al2p = jnp.zeros((DP,), _f32).at[:D_OUT].set(al2)
    ar2p = jnp.zeros((DP,), _f32).at[:D_OUT].set(ar2)

    htab1, el1, er1 = _tc_head1(features, w1p, al1p, ar1p)
    num1 = _sc_edge(ei4, htab1, el1, er1).reshape(NC, N, DP)
    htab2, el2, er2 = _tc_mid(num1, b1p, w2p, al2p, ar2p)
    num2 = _sc_edge(ei4, htab2, el2, er2).reshape(NC, N, DP)
    return _tc_final(num2, b2)
